# baseline (speedup 1.0000x reference)
import functools

import jax
import jax.numpy as jnp
from jax import lax
from jax.experimental import pallas as pl
from jax.experimental.pallas import tpu as pltpu
from jax.experimental.pallas import tpu_sc as plsc

N_NODES = 10000
HID = 300
SLAB = 128
N_LAYERS = 5

CHUNK = 128
N_SUBCORES = 16
N_CORES = 2
ROWS_PER_TILE = 632
ROWS_LAST = N_NODES - ROWS_PER_TILE * (N_SUBCORES - 1)
AGG_ROWS = N_NODES + 8


def _make_agg_kernel(n_slabs, chunks_per_worker):
    mesh = plsc.VectorSubcoreMesh(core_axis_name="c", subcore_axis_name="s")
    cpw0, cpw1 = chunks_per_worker
    ct = cpw0 + cpw1
    shares = [(cpw0, cpw1)] + [(ct, 0)] * (n_slabs - 1)
    out_rows = (n_slabs + (1 if cpw1 else 0)) * N_NODES

    def _pieces(cpw):
        out, rem = [], cpw
        while rem:
            p = next(q for q in (32, 24, 16, 8) if q <= rem)
            out.append(p)
            rem -= p
        assert sum(out) == cpw and all(p % 8 == 0 for p in out)
        return out

    piece_max = 32

    @functools.partial(
        pl.kernel,
        mesh=mesh,
        out_type=jax.ShapeDtypeStruct((out_rows, SLAB), jnp.float32),
        scratch_types=[
            pltpu.VMEM((piece_max, CHUNK), jnp.int32),
            pltpu.VMEM((piece_max, CHUNK), jnp.int32),
            pltpu.VMEM((CHUNK, SLAB), jnp.float32),
            pltpu.VMEM((CHUNK, SLAB), jnp.float32),
            pltpu.VMEM((64, SLAB), jnp.float32),
            pltpu.VMEM_SHARED((AGG_ROWS, SLAB), jnp.float32),
            pltpu.SemaphoreType.DMA,
            pltpu.SemaphoreType.DMA,
        ],
    )
    def agg_kernel(h_hbm, src_hbm, dst_hbm, zeros_hbm, out_hbm,
                   src_v, dst_v, gbuf0, gbuf1, zbuf, acc, sem0, sem1):
        c = lax.axis_index("c")
        s = lax.axis_index("s")

        gbufs = (gbuf0, gbuf1)
        sems = (sem0, sem1)

        def make_gather(table, local, b):
            return pltpu.make_async_copy(
                table.at[src_v.at[local]], gbufs[b], sems[b])

        def scatter(local, b):
            pltpu.sync_copy(gbufs[b], acc.at[dst_v.at[local]], add=True)

        def run_range(table, worker_base, pieces):
            done = 0
            for piece in pieces:
                base_chunk = worker_base + done
                done += piece
                pltpu.sync_copy(src_hbm.at[pl.ds(base_chunk, piece)],
                                src_v.at[pl.ds(0, piece)])
                pltpu.sync_copy(dst_hbm.at[pl.ds(base_chunk, piece)],
                                dst_v.at[pl.ds(0, piece)])

                make_gather(table, 0, 0).start()
                make_gather(table, 1, 1).start()

                def body(t, carry):
                    k = t * 2
                    make_gather(table, k, 0).wait()
                    scatter(k, 0)
                    make_gather(table, k + 2, 0).start()
                    make_gather(table, k + 1, 1).wait()
                    scatter(k + 1, 1)
                    make_gather(table, k + 3, 1).start()
                    return carry

                lax.fori_loop(0, piece // 2 - 1, body, 0)
                make_gather(table, piece - 2, 0).wait()
                scatter(piece - 2, 0)
                make_gather(table, piece - 1, 1).wait()
                scatter(piece - 1, 1)

        pltpu.sync_copy(zeros_hbm.at[pl.ds(0, 64)], zbuf)

        def zero_rows(base, rows):
            full, rem = rows // 64, rows % 64
            for i in range(full):
                pltpu.sync_copy(zbuf, acc.at[pl.ds(base + i * 64, 64)])
            if rem:
                pltpu.sync_copy(zbuf.at[pl.ds(0, rem)],
                                acc.at[pl.ds(base + full * 64, rem)])

        for slab in range(n_slabs):
            table = h_hbm.at[slab]
            s_cpw0, s_cpw1 = shares[slab]
            core_active = (c < N_CORES) if s_cpw1 else (c == 0)

            @pl.when(core_active & (s < N_SUBCORES - 1))
            def _zero_main():
                zero_rows(s * ROWS_PER_TILE, ROWS_PER_TILE)

            @pl.when(core_active & (s == N_SUBCORES - 1))
            def _zero_tail():
                zero_rows((N_SUBCORES - 1) * ROWS_PER_TILE, ROWS_LAST + 8)

            plsc.subcore_barrier()

            @pl.when(c == 0)
            def _core0():
                run_range(table, s * s_cpw0, _pieces(s_cpw0))

            if s_cpw1:
                @pl.when(c == 1)
                def _core1():
                    run_range(table, N_SUBCORES * s_cpw0 + s * s_cpw1,
                              _pieces(s_cpw1))

            plsc.subcore_barrier()

            out_slab = jnp.where(c == 0, slab, n_slabs)
            out_base = out_slab * N_NODES + s * ROWS_PER_TILE

            @pl.when(core_active & (s < N_SUBCORES - 1))
            def _copy_main():
                pltpu.sync_copy(
                    acc.at[pl.ds(s * ROWS_PER_TILE, ROWS_PER_TILE)],
                    out_hbm.at[pl.ds(out_base, ROWS_PER_TILE)])

            @pl.when(core_active & (s == N_SUBCORES - 1))
            def _copy_tail():
                pltpu.sync_copy(
                    acc.at[pl.ds((N_SUBCORES - 1) * ROWS_PER_TILE, ROWS_LAST)],
                    out_hbm.at[pl.ds(out_base, ROWS_LAST)])

            plsc.subcore_barrier()

    return agg_kernel


def _make_mlp_kernel(n_slabs_in, n_slabs_out, last, bn, has_extra):
    g = N_NODES // bn
    if last:
        out_shape = jax.ShapeDtypeStruct((N_NODES, HID), jnp.float32)
        out_spec = pl.BlockSpec((bn, HID), lambda i: (i, 0))
        out_cols = HID
    else:
        out_shape = jax.ShapeDtypeStruct((n_slabs_out, N_NODES, SLAB), jnp.float32)
        out_spec = pl.BlockSpec((n_slabs_out, bn, SLAB), lambda i: (0, i, 0))
        out_cols = n_slabs_out * SLAB

    def mlp_kernel(h_ref, a_ref, *rest):
        if has_extra:
            a1_ref, w1_ref, b1_ref, w2_ref, b2_ref, o_ref = rest
        else:
            w1_ref, b1_ref, w2_ref, b2_ref, o_ref = rest
            a1_ref = None
        t = b1_ref[...]
        for k in range(n_slabs_in):
            m = h_ref[k] + a_ref[k]
            if k == 0 and a1_ref is not None:
                m = m + a1_ref[...]
            t = t + jnp.dot(m, w1_ref[k], preferred_element_type=jnp.float32)
        t = jnp.maximum(t, 0.0)
        o = jnp.dot(t, w2_ref[...], preferred_element_type=jnp.float32) + b2_ref[...]
        if last:
            o_ref[...] = o
        else:
            o = jnp.maximum(o, 0.0)
            for k in range(n_slabs_out):
                o_ref[k] = o[:, k * SLAB:(k + 1) * SLAB]

    in_specs = [
        pl.BlockSpec((n_slabs_in, bn, SLAB), lambda i: (0, i, 0)),
        pl.BlockSpec((n_slabs_in, bn, SLAB), lambda i: (0, i, 0)),
    ]
    if has_extra:
        in_specs.append(pl.BlockSpec((bn, SLAB), lambda i: (i, 0)))
    in_specs += [
        pl.BlockSpec((n_slabs_in, SLAB, 2 * HID), lambda i: (0, 0, 0)),
        pl.BlockSpec((1, 2 * HID), lambda i: (0, 0)),
        pl.BlockSpec((2 * HID, out_cols), lambda i: (0, 0)),
        pl.BlockSpec((1, out_cols), lambda i: (0, 0)),
    ]
    return pl.pallas_call(
        mlp_kernel,
        grid=(g,),
        in_specs=in_specs,
        out_specs=out_spec,
        out_shape=out_shape,
    )


def kernel(x, edge_index, batch, params):
    n, f_in = x.shape
    e = edge_index.shape[1]
    assert f_in == SLAB and n == N_NODES
    n_slabs_h = -(-HID // SLAB)

    unit = 4 * CHUNK * N_SUBCORES * N_CORES
    ep = -(-e // unit) * unit
    pad = ep - e
    src = jnp.concatenate([edge_index[0], jnp.zeros((pad,), jnp.int32)])
    dst = jnp.concatenate([edge_index[1], jnp.full((pad,), n, jnp.int32)])
    src = src.reshape(ep // CHUNK, CHUNK)
    dst = dst.reshape(ep // CHUNK, CHUNK)
    ct_per_tile = ep // (CHUNK * N_SUBCORES)
    cpw0 = ct_per_tile - 8
    cpw1 = ct_per_tile - cpw0

    chunks_per_worker = (cpw0, cpw1)

    zeros_hbm = jnp.zeros((ROWS_PER_TILE, SLAB), jnp.float32)

    agg1 = _make_agg_kernel(1, chunks_per_worker)
    agg3 = _make_agg_kernel(n_slabs_h, chunks_per_worker)

    h = x.reshape(1, n, SLAB)
    for l in range(N_LAYERS):
        n_slabs_in = 1 if l == 0 else n_slabs_h
        last = l == N_LAYERS - 1

        scale = params['bn_g_%d' % l] / jnp.sqrt(1.0 + 1e-05)
        w2 = params['W2_%d' % l] * scale[None, :]
        b2 = params['b2_%d' % l] * scale + params['bn_b_%d' % l]

        w1 = params['W1_%d' % l]
        din = w1.shape[0]
        if n_slabs_in * SLAB > din:
            w1 = jnp.concatenate(
                [w1, jnp.zeros((n_slabs_in * SLAB - din, 2 * HID), jnp.float32)])
        w1 = w1.reshape(n_slabs_in, SLAB, 2 * HID)
        if not last:
            out_cols = n_slabs_h * SLAB
            w2 = jnp.concatenate(
                [w2, jnp.zeros((2 * HID, out_cols - HID), jnp.float32)], axis=1)
            b2 = jnp.concatenate([b2, jnp.zeros((out_cols - HID,), jnp.float32)])
        b1 = params['b1_%d' % l].reshape(1, 2 * HID)
        b2 = b2.reshape(1, -1)

        agg_fn = agg1 if l == 0 else agg3
        agg_flat = agg_fn(h, src, dst, zeros_hbm)
        agg0 = agg_flat[:n_slabs_in * n].reshape(n_slabs_in, n, SLAB)

        mlp = _make_mlp_kernel(n_slabs_in, n_slabs_h, last, 2000, cpw1 > 0)
        if cpw1 > 0:
            agg_extra = agg_flat[n_slabs_in * n:]
            h = mlp(h, agg0, agg_extra, w1, b1, w2, b2)
        else:
            h = mlp(h, agg0, w1, b1, w2, b2)

    return h


# speedup vs baseline: 1.2426x; 1.2426x over previous
import functools

import jax
import jax.numpy as jnp
from jax import lax
from jax.experimental import pallas as pl
from jax.experimental.pallas import tpu as pltpu
from jax.experimental.pallas import tpu_sc as plsc

N_NODES = 10000
HID = 300
SLAB = 128
N_LAYERS = 5

CHUNK = 128
N_SUBCORES = 16
N_CORES = 2
ROWS_PER_TILE = 632
ROWS_LAST = N_NODES - ROWS_PER_TILE * (N_SUBCORES - 1)
AGG_ROWS = N_NODES + 8


def _make_agg_kernel(n_slabs, chunks_per_worker):
    mesh = plsc.VectorSubcoreMesh(core_axis_name="c", subcore_axis_name="s")
    cpw0, cpw1 = chunks_per_worker
    n_partials = 2 if cpw1 else 1

    def _pieces(cpw):
        out, rem = [], cpw
        while rem:
            p = next(q for q in (32, 24, 16, 8) if q <= rem)
            out.append(p)
            rem -= p
        assert sum(out) == cpw and all(p % 8 == 0 for p in out)
        return out

    pieces0 = _pieces(cpw0)
    pieces1 = _pieces(cpw1) if cpw1 else []
    piece_max = max(pieces0 + pieces1)

    @functools.partial(
        pl.kernel,
        mesh=mesh,
        out_type=jax.ShapeDtypeStruct(
            (n_partials * n_slabs * N_NODES, SLAB), jnp.float32),
        scratch_types=[
            pltpu.VMEM((piece_max, CHUNK), jnp.int32),
            pltpu.VMEM((piece_max, CHUNK), jnp.int32),
            pltpu.VMEM((CHUNK, SLAB), jnp.float32),
            pltpu.VMEM((CHUNK, SLAB), jnp.float32),
            pltpu.VMEM((64, SLAB), jnp.float32),
            pltpu.VMEM_SHARED((AGG_ROWS, SLAB), jnp.float32),
            pltpu.SemaphoreType.DMA,
            pltpu.SemaphoreType.DMA,
        ],
    )
    def agg_kernel(h_hbm, src_hbm, dst_hbm, zeros_hbm, out_hbm,
                   src_v, dst_v, gbuf0, gbuf1, zbuf, acc, sem0, sem1):
        c = lax.axis_index("c")
        s = lax.axis_index("s")

        gbufs = (gbuf0, gbuf1)
        sems = (sem0, sem1)

        def make_gather(table, local, b):
            return pltpu.make_async_copy(
                table.at[src_v.at[local]], gbufs[b], sems[b])

        def scatter(local, b):
            pltpu.sync_copy(gbufs[b], acc.at[dst_v.at[local]], add=True)

        def run_range(table, worker_base, pieces):
            done = 0
            for piece in pieces:
                base_chunk = worker_base + done
                done += piece
                pltpu.sync_copy(src_hbm.at[pl.ds(base_chunk, piece)],
                                src_v.at[pl.ds(0, piece)])
                pltpu.sync_copy(dst_hbm.at[pl.ds(base_chunk, piece)],
                                dst_v.at[pl.ds(0, piece)])

                make_gather(table, 0, 0).start()
                make_gather(table, 1, 1).start()

                def body(t, carry):
                    k = t * 2
                    make_gather(table, k, 0).wait()
                    scatter(k, 0)
                    make_gather(table, k + 2, 0).start()
                    make_gather(table, k + 1, 1).wait()
                    scatter(k + 1, 1)
                    make_gather(table, k + 3, 1).start()
                    return carry

                lax.fori_loop(0, piece // 2 - 1, body, 0)
                make_gather(table, piece - 2, 0).wait()
                scatter(piece - 2, 0)
                make_gather(table, piece - 1, 1).wait()
                scatter(piece - 1, 1)

        core_active = (c == 0) if n_partials == 1 else (c < N_CORES)

        pltpu.sync_copy(zeros_hbm.at[pl.ds(0, 64)], zbuf)

        def zero_rows(base, rows):
            full, rem = rows // 64, rows % 64
            for i in range(full):
                pltpu.sync_copy(zbuf, acc.at[pl.ds(base + i * 64, 64)])
            if rem:
                pltpu.sync_copy(zbuf.at[pl.ds(0, rem)],
                                acc.at[pl.ds(base + full * 64, rem)])

        for slab in range(n_slabs):
            table = h_hbm.at[slab]
            @pl.when(core_active & (s < N_SUBCORES - 1))
            def _zero_main():
                zero_rows(s * ROWS_PER_TILE, ROWS_PER_TILE)

            @pl.when(core_active & (s == N_SUBCORES - 1))
            def _zero_tail():
                zero_rows((N_SUBCORES - 1) * ROWS_PER_TILE, ROWS_LAST + 8)

            plsc.subcore_barrier()

            @pl.when(c == 0)
            def _core0():
                run_range(table, s * cpw0, pieces0)

            if cpw1:
                @pl.when(c == 1)
                def _core1():
                    run_range(table, N_SUBCORES * cpw0 + s * cpw1, pieces1)

            plsc.subcore_barrier()

            out_base = (c * n_slabs + slab) * N_NODES + s * ROWS_PER_TILE

            @pl.when(core_active & (s < N_SUBCORES - 1))
            def _copy_main():
                pltpu.sync_copy(
                    acc.at[pl.ds(s * ROWS_PER_TILE, ROWS_PER_TILE)],
                    out_hbm.at[pl.ds(out_base, ROWS_PER_TILE)])

            @pl.when(core_active & (s == N_SUBCORES - 1))
            def _copy_tail():
                pltpu.sync_copy(
                    acc.at[pl.ds((N_SUBCORES - 1) * ROWS_PER_TILE, ROWS_LAST)],
                    out_hbm.at[pl.ds(out_base, ROWS_LAST)])

            plsc.subcore_barrier()

    return agg_kernel


def _make_mlp_kernel(n_slabs_in, n_slabs_out, last, bn, n_partials):
    g = N_NODES // bn
    if last:
        out_shape = jax.ShapeDtypeStruct((N_NODES, HID), jnp.float32)
        out_spec = pl.BlockSpec((bn, HID), lambda i: (i, 0))
        out_cols = HID
    else:
        out_shape = jax.ShapeDtypeStruct((n_slabs_out, N_NODES, SLAB), jnp.float32)
        out_spec = pl.BlockSpec((n_slabs_out, bn, SLAB), lambda i: (0, i, 0))
        out_cols = n_slabs_out * SLAB

    def mlp_kernel(h_ref, a_ref, w1_ref, b1_ref, w2_ref, b2_ref, o_ref):
        t = b1_ref[...]
        for k in range(n_slabs_in):
            m = h_ref[k]
            for p in range(n_partials):
                m = m + a_ref[p, k]
            t = t + jnp.dot(m, w1_ref[k], preferred_element_type=jnp.float32)
        t = jnp.maximum(t, 0.0)
        o = jnp.dot(t, w2_ref[...], preferred_element_type=jnp.float32) + b2_ref[...]
        if last:
            o_ref[...] = o
        else:
            o = jnp.maximum(o, 0.0)
            for k in range(n_slabs_out):
                o_ref[k] = o[:, k * SLAB:(k + 1) * SLAB]

    return pl.pallas_call(
        mlp_kernel,
        grid=(g,),
        in_specs=[
            pl.BlockSpec((n_slabs_in, bn, SLAB), lambda i: (0, i, 0)),
            pl.BlockSpec((n_partials, n_slabs_in, bn, SLAB),
                         lambda i: (0, 0, i, 0)),
            pl.BlockSpec((n_slabs_in, SLAB, 2 * HID), lambda i: (0, 0, 0)),
            pl.BlockSpec((1, 2 * HID), lambda i: (0, 0)),
            pl.BlockSpec((2 * HID, out_cols), lambda i: (0, 0)),
            pl.BlockSpec((1, out_cols), lambda i: (0, 0)),
        ],
        out_specs=out_spec,
        out_shape=out_shape,
    )


def kernel(x, edge_index, batch, params):
    n, f_in = x.shape
    e = edge_index.shape[1]
    assert f_in == SLAB and n == N_NODES
    n_slabs_h = -(-HID // SLAB)

    unit = 4 * CHUNK * N_SUBCORES * N_CORES
    ep = -(-e // unit) * unit
    pad = ep - e
    src = jnp.concatenate([edge_index[0], jnp.zeros((pad,), jnp.int32)])
    dst = jnp.concatenate([edge_index[1], jnp.full((pad,), n, jnp.int32)])
    src = src.reshape(ep // CHUNK, CHUNK)
    dst = dst.reshape(ep // CHUNK, CHUNK)
    ct_per_tile = ep // (CHUNK * N_SUBCORES)
    cpw0 = ct_per_tile - 8
    cpw1 = ct_per_tile - cpw0
    n_partials = 2 if cpw1 else 1
    chunks_per_worker = (cpw0, cpw1)

    zeros_hbm = jnp.zeros((ROWS_PER_TILE, SLAB), jnp.float32)

    agg1 = _make_agg_kernel(1, chunks_per_worker)
    agg3 = _make_agg_kernel(n_slabs_h, chunks_per_worker)

    h = x.reshape(1, n, SLAB)
    for l in range(N_LAYERS):
        n_slabs_in = 1 if l == 0 else n_slabs_h
        last = l == N_LAYERS - 1

        scale = params['bn_g_%d' % l] / jnp.sqrt(1.0 + 1e-05)
        w2 = params['W2_%d' % l] * scale[None, :]
        b2 = params['b2_%d' % l] * scale + params['bn_b_%d' % l]

        w1 = params['W1_%d' % l]
        din = w1.shape[0]
        if n_slabs_in * SLAB > din:
            w1 = jnp.concatenate(
                [w1, jnp.zeros((n_slabs_in * SLAB - din, 2 * HID), jnp.float32)])
        w1 = w1.reshape(n_slabs_in, SLAB, 2 * HID)
        if not last:
            out_cols = n_slabs_h * SLAB
            w2 = jnp.concatenate(
                [w2, jnp.zeros((2 * HID, out_cols - HID), jnp.float32)], axis=1)
            b2 = jnp.concatenate([b2, jnp.zeros((out_cols - HID,), jnp.float32)])
        b1 = params['b1_%d' % l].reshape(1, 2 * HID)
        b2 = b2.reshape(1, -1)

        agg_fn = agg1 if l == 0 else agg3
        agg_flat = agg_fn(h, src, dst, zeros_hbm)
        agg = agg_flat.reshape(n_partials, n_slabs_in, n, SLAB)

        mlp = _make_mlp_kernel(n_slabs_in, n_slabs_h, last, 2000, n_partials)
        h = mlp(h, agg, w1, b1, w2, b2)

    return h
